# baseline (speedup 1.0000x reference)
import functools
import math

import jax
import jax.numpy as jnp
import numpy as np
from jax import lax
from jax.experimental import pallas as pl
from jax.experimental.pallas import tpu as pltpu
from jax.experimental.pallas import tpu_sc as plsc

VOCAB = 50267
EMBED_DIM = 128
SEQ_LEN = 2048


def _positional_encoding_np(seq_len: int, d: int) -> np.ndarray:
    position = np.arange(seq_len, dtype=np.float32)[:, None]
    div_term = np.exp(
        np.arange(0, d, 2, dtype=np.float32) * (-math.log(10000.0) / d)
    ).astype(np.float32)
    pe = np.zeros((seq_len, d), dtype=np.float32)
    pe[:, 0::2] = np.sin(position * div_term)
    pe[:, 1::2] = np.cos(position * div_term)
    return pe


_PE = _positional_encoding_np(SEQ_LEN, EMBED_DIM)

_CHUNKS = (8, 24, 40, 56)
_NCHUNK = len(_CHUNKS)


@functools.lru_cache(maxsize=None)
def _build_sc_kernel():
    info = plsc.get_sparse_core_info()
    NC, NS, L = 1, info.num_subcores, info.num_lanes
    NW = NC * NS
    BPW = SEQ_LEN // NW

    mesh = plsc.VectorSubcoreMesh(
        core_axis_name="c", subcore_axis_name="s", num_cores=NC)

    @functools.partial(
        pl.kernel,
        mesh=mesh,
        out_type=jax.ShapeDtypeStruct((1, SEQ_LEN, EMBED_DIM), jnp.float32),
        scratch_types=[
            pltpu.VMEM((BPW,), jnp.int32),
            pltpu.VMEM((BPW, EMBED_DIM), jnp.float32),
            pltpu.SemaphoreType.DMA,
        ]
        + [pltpu.SemaphoreType.DMA] * (3 * _NCHUNK),
    )
    def emb_kernel(idx_hbm, table_hbm, pe_hbm, out_hbm, idx_v, rows_v,
                   sem_i, *sems):
        sem_p = sems[:_NCHUNK]
        sem_g = sems[_NCHUNK:2 * _NCHUNK]
        sem_o = sems[2 * _NCHUNK:]
        wid = lax.axis_index("s") * NC + lax.axis_index("c")
        base = wid * BPW
        offs = [sum(_CHUNKS[:c]) for c in range(_NCHUNK)]
        ci = pltpu.async_copy(idx_hbm.at[pl.ds(base, BPW)], idx_v, sem_i)
        cps = [
            pltpu.async_copy(pe_hbm.at[pl.ds(base + offs[c], _CHUNKS[c])],
                             rows_v.at[pl.ds(offs[c], _CHUNKS[c])], sem_p[c])
            for c in range(_NCHUNK)
        ]
        ci.wait()
        gs = []
        for c in range(_NCHUNK):
            cps[c].wait()
            gs.append(pltpu.async_copy(
                table_hbm.at[idx_v.at[pl.ds(offs[c], _CHUNKS[c])]],
                rows_v.at[pl.ds(offs[c], _CHUNKS[c])], sem_g[c], add=True))
        os_ = []
        for c in range(_NCHUNK):
            gs[c].wait()
            os_.append(pltpu.async_copy(
                rows_v.at[pl.ds(offs[c], _CHUNKS[c])],
                out_hbm.at[0, pl.ds(base + offs[c], _CHUNKS[c])], sem_o[c]))
        for c in range(_NCHUNK):
            os_[c].wait()

    return emb_kernel


def kernel(token_id, table):
    emb = _build_sc_kernel()
    pe = jnp.asarray(_PE)
    return emb(token_id.astype(jnp.int32), table, pe)


# speedup vs baseline: 1.0083x; 1.0083x over previous
import functools
import math

import jax
import jax.numpy as jnp
import numpy as np
from jax import lax
from jax.experimental import pallas as pl
from jax.experimental.pallas import tpu as pltpu
from jax.experimental.pallas import tpu_sc as plsc

VOCAB = 50267
EMBED_DIM = 128
SEQ_LEN = 2048


def _positional_encoding_np(seq_len: int, d: int) -> np.ndarray:
    position = np.arange(seq_len, dtype=np.float32)[:, None]
    div_term = np.exp(
        np.arange(0, d, 2, dtype=np.float32) * (-math.log(10000.0) / d)
    ).astype(np.float32)
    pe = np.zeros((seq_len, d), dtype=np.float32)
    pe[:, 0::2] = np.sin(position * div_term)
    pe[:, 1::2] = np.cos(position * div_term)
    return pe


_PE = _positional_encoding_np(SEQ_LEN, EMBED_DIM)

_CHUNKS = (16, 32, 80)
_NCHUNK = len(_CHUNKS)


@functools.lru_cache(maxsize=None)
def _build_sc_kernel():
    info = plsc.get_sparse_core_info()
    NC, NS, L = 1, info.num_subcores, info.num_lanes
    NW = NC * NS
    BPW = SEQ_LEN // NW

    mesh = plsc.VectorSubcoreMesh(
        core_axis_name="c", subcore_axis_name="s", num_cores=NC)

    @functools.partial(
        pl.kernel,
        mesh=mesh,
        out_type=jax.ShapeDtypeStruct((1, SEQ_LEN, EMBED_DIM), jnp.float32),
        scratch_types=[
            pltpu.VMEM((BPW,), jnp.int32),
            pltpu.VMEM((BPW, EMBED_DIM), jnp.float32),
            pltpu.SemaphoreType.DMA,
        ]
        + [pltpu.SemaphoreType.DMA] * (3 * _NCHUNK),
    )
    def emb_kernel(idx_hbm, table_hbm, pe_hbm, out_hbm, idx_v, rows_v,
                   sem_i, *sems):
        sem_p = sems[:_NCHUNK]
        sem_g = sems[_NCHUNK:2 * _NCHUNK]
        sem_o = sems[2 * _NCHUNK:]
        wid = lax.axis_index("s") * NC + lax.axis_index("c")
        base = wid * BPW
        offs = [sum(_CHUNKS[:c]) for c in range(_NCHUNK)]
        ci = pltpu.async_copy(idx_hbm.at[pl.ds(base, BPW)], idx_v, sem_i)
        cps = [
            pltpu.async_copy(pe_hbm.at[pl.ds(base + offs[c], _CHUNKS[c])],
                             rows_v.at[pl.ds(offs[c], _CHUNKS[c])], sem_p[c])
            for c in range(_NCHUNK)
        ]
        ci.wait()
        gs = []
        for c in range(_NCHUNK):
            cps[c].wait()
            gs.append(pltpu.async_copy(
                table_hbm.at[idx_v.at[pl.ds(offs[c], _CHUNKS[c])]],
                rows_v.at[pl.ds(offs[c], _CHUNKS[c])], sem_g[c], add=True))
        os_ = []
        for c in range(_NCHUNK):
            gs[c].wait()
            os_.append(pltpu.async_copy(
                rows_v.at[pl.ds(offs[c], _CHUNKS[c])],
                out_hbm.at[0, pl.ds(base + offs[c], _CHUNKS[c])], sem_o[c]))
        for c in range(_NCHUNK):
            os_[c].wait()

    return emb_kernel


def kernel(token_id, table):
    emb = _build_sc_kernel()
    pe = jnp.asarray(_PE)
    return emb(token_id.astype(jnp.int32), table, pe)
